# baseline (speedup 1.0000x reference)
import functools

import jax
import jax.numpy as jnp
from jax import lax
from jax.experimental import pallas as pl
from jax.experimental.pallas import tpu as pltpu
from jax.experimental.pallas import tpu_sc as plsc

EMB = 32
NROW = 1000000
NROWB = 999936
NTAIL = NROW - NROWB
BATCH = 16384
NUM_AGE = 7
NUM_GENDERS = 2
NUM_OCC = 21
NUM_GENRES = 18
HID = 128

_NC = 2
_NS = 16
_NW = _NC * _NS
_CHUNK = 128


def _flatten_body(tblT_hbm, flat_hbm, sem):
    copies = [
        pltpu.make_async_copy(
            tblT_hbm.at[d, pl.ds(0, NROWB)],
            flat_hbm.at[pl.ds(d * NROWB, NROWB)],
            sem,
        )
        for d in range(EMB)
    ]
    for c in copies:
        c.start()
    for c in copies:
        c.wait()


def _flatten(tableT):
    return pl.pallas_call(
        _flatten_body,
        in_specs=[pl.BlockSpec(memory_space=pl.ANY)],
        out_specs=pl.BlockSpec(memory_space=pl.ANY),
        out_shape=jax.ShapeDtypeStruct((EMB * NROWB,), jnp.float32),
        scratch_shapes=[pltpu.SemaphoreType.DMA],
    )(tableT)


def _gather_body(flat_hbm, idx_hbm, outT_hbm, idx_v, vals_v, sem):
    wid = lax.axis_index("s") * _NC + lax.axis_index("c")
    pltpu.sync_copy(idx_hbm.at[wid], idx_v)
    copies = [
        pltpu.async_copy(
            flat_hbm.at[idx_v.at[pl.ds(j * _CHUNK, _CHUNK)]],
            vals_v.at[pl.ds(j * _CHUNK, _CHUNK)],
            sem,
        )
        for j in range(BATCH // _CHUNK)
    ]
    for c in copies:
        c.wait()
    pltpu.sync_copy(vals_v, outT_hbm.at[wid])


def _movie_gather(flat, idx_elems):
    mesh = plsc.VectorSubcoreMesh(core_axis_name="c", subcore_axis_name="s")
    k = pl.kernel(
        _gather_body,
        mesh=mesh,
        out_type=jax.ShapeDtypeStruct((EMB, BATCH), jnp.float32),
        scratch_types=[
            pltpu.VMEM((BATCH,), jnp.int32),
            pltpu.VMEM((BATCH,), jnp.float32),
            pltpu.SemaphoreType.DMA,
        ],
    )
    return k(flat, idx_elems)


_B_BLK = 2048
_GRID = BATCH // _B_BLK


def _mlp_body(movT_ref, genT_ref, mvi_ref, age_ref, gdr_ref, occ_ref,
              tailt_ref, aget_ref, gdrt_ref, occt_ref, gw_ref, gb_ref,
              w1_ref, b1_ref, w2_ref, b2_ref, out_ref):
    f32 = jnp.float32
    W1 = w1_ref[...]
    Wa = W1[0:32]
    Wm = W1[32:64]
    Wg = W1[64:96]
    Wo = W1[96:128]
    Wgen = W1[128:160]

    def onehot_t(idx_row, n):
        iota = lax.broadcasted_iota(jnp.int32, (n, _B_BLK), 0)
        return (idx_row == iota).astype(f32)

    dot = functools.partial(jnp.dot, preferred_element_type=f32)

    def tdot(lhs_t, rhs):
        return lax.dot_general(
            lhs_t, rhs, (((0,), (0,)), ((), ())),
            preferred_element_type=f32,
        )

    mvi = mvi_ref[...]
    movT = jnp.where(mvi >= NROWB, 0.0, movT_ref[...])
    acc = tdot(movT, Wm)
    tail_iota = lax.broadcasted_iota(jnp.int32, (NTAIL, _B_BLK), 0)
    tail_oh = (mvi - NROWB == tail_iota).astype(f32)
    acc += tdot(tail_oh, dot(tailt_ref[...], Wm))
    acc += tdot(onehot_t(age_ref[...], NUM_AGE), dot(aget_ref[...], Wa))
    acc += tdot(onehot_t(gdr_ref[...], NUM_GENDERS), dot(gdrt_ref[...], Wg))
    acc += tdot(onehot_t(occ_ref[...], NUM_OCC), dot(occt_ref[...], Wo))
    acc += tdot(genT_ref[...], dot(gw_ref[...], Wgen))
    acc += dot(gb_ref[...], Wgen) + b1_ref[...]
    h = jnp.maximum(acc, 0.0)
    out_ref[...] = lax.dot_general(
        w2_ref[...], h, (((1,), (1,)), ((), ())),
        preferred_element_type=f32,
    ) + b2_ref[...]


def _mlp(movT, genresT, mvi_r, age_r, gdr_r, occ_r,
         tail_table, age_table, gender_table, occupation_table,
         genre_W, genre_b, fc1_W, fc1_b, fc2_W, fc2_b,
         interpret=False):
    row_spec = lambda d: pl.BlockSpec((d, _B_BLK), lambda i: (0, i))
    full_spec = lambda a, b: pl.BlockSpec((a, b), lambda i: (0, 0))
    out2 = pl.pallas_call(
        _mlp_body,
        grid=(_GRID,),
        in_specs=[
            row_spec(EMB),
            row_spec(NUM_GENRES),
            row_spec(1),
            row_spec(1),
            row_spec(1),
            row_spec(1),
            full_spec(NTAIL, EMB),
            full_spec(NUM_AGE, EMB),
            full_spec(NUM_GENDERS, EMB),
            full_spec(NUM_OCC, EMB),
            full_spec(NUM_GENRES, EMB),
            full_spec(1, EMB),
            full_spec(5 * EMB, HID),
            full_spec(1, HID),
            full_spec(1, HID),
            full_spec(1, 1),
        ],
        out_specs=pl.BlockSpec((1, _B_BLK), lambda i: (0, i)),
        out_shape=jax.ShapeDtypeStruct((1, BATCH), jnp.float32),
        interpret=interpret,
    )(movT, genresT, mvi_r, age_r, gdr_r, occ_r,
      tail_table, age_table, gender_table, occupation_table,
      genre_W, genre_b.reshape(1, EMB),
      fc1_W, fc1_b.reshape(1, HID), fc2_W.reshape(1, HID),
      fc2_b.reshape(1, 1))
    return out2.reshape(BATCH)


def kernel(age_group, movie, gender, occupation, genres,
           movie_table, gender_table, occupation_table, age_table,
           genre_W, genre_b, fc1_W, fc1_b, fc2_W, fc2_b):
    i32 = jnp.int32
    movie = movie.astype(i32)
    flat = _flatten(movie_table.T)
    movie_c = jnp.minimum(movie, NROWB - 1)
    idx_elems = (jnp.arange(EMB, dtype=i32) * NROWB)[:, None] + movie_c[None, :]
    movT = _movie_gather(flat, idx_elems)
    tail_table = lax.slice(movie_table, (NROWB, 0), (NROW, EMB))
    return _mlp(
        movT,
        genres.astype(jnp.float32).T,
        movie.reshape(1, BATCH),
        age_group.astype(i32).reshape(1, BATCH),
        gender.astype(i32).reshape(1, BATCH),
        occupation.astype(i32).reshape(1, BATCH),
        tail_table, age_table, gender_table, occupation_table,
        genre_W, genre_b, fc1_W, fc1_b, fc2_W, fc2_b,
    )


# speedup vs baseline: 1.6602x; 1.6602x over previous
import functools

import jax
import jax.numpy as jnp
from jax import lax
from jax.experimental import pallas as pl
from jax.experimental.pallas import tpu as pltpu
from jax.experimental.pallas import tpu_sc as plsc

EMB = 32
NROW = 1000000
NROWB = 999936
NTAIL = NROW - NROWB
BATCH = 16384
NUM_AGE = 7
NUM_GENDERS = 2
NUM_OCC = 21
NUM_GENRES = 18
HID = 128

_NC = 2
_NS = 16
_NW = _NC * _NS
_CHUNK = 128


_CP_STEPS = 42
_CP_L = NROWB // _CP_STEPS


def _flatten_body(tblT_ref, out_ref):
    out_ref[...] = tblT_ref[...]


def _flatten(tableT):
    out = pl.pallas_call(
        _flatten_body,
        grid=(_CP_STEPS,),
        in_specs=[pl.BlockSpec((EMB, _CP_L), lambda i: (0, i))],
        out_specs=pl.BlockSpec((EMB, _CP_L), lambda i: (0, i)),
        out_shape=jax.ShapeDtypeStruct((EMB, NROWB), jnp.float32),
    )(tableT)
    return out.reshape(EMB * NROWB)


def _gather_body(flat_hbm, idx_hbm, outT_hbm, idx_v, vals_v, sem):
    wid = lax.axis_index("s") * _NC + lax.axis_index("c")
    pltpu.sync_copy(idx_hbm.at[wid], idx_v)
    copies = [
        pltpu.async_copy(
            flat_hbm.at[idx_v.at[pl.ds(j * _CHUNK, _CHUNK)]],
            vals_v.at[pl.ds(j * _CHUNK, _CHUNK)],
            sem,
        )
        for j in range(BATCH // _CHUNK)
    ]
    for c in copies:
        c.wait()
    pltpu.sync_copy(vals_v, outT_hbm.at[wid])


def _movie_gather(flat, idx_elems):
    mesh = plsc.VectorSubcoreMesh(core_axis_name="c", subcore_axis_name="s")
    k = pl.kernel(
        _gather_body,
        mesh=mesh,
        out_type=jax.ShapeDtypeStruct((EMB, BATCH), jnp.float32),
        scratch_types=[
            pltpu.VMEM((BATCH,), jnp.int32),
            pltpu.VMEM((BATCH,), jnp.float32),
            pltpu.SemaphoreType.DMA,
        ],
    )
    return k(flat, idx_elems)


_B_BLK = 2048
_GRID = BATCH // _B_BLK


def _mlp_body(movT_ref, genT_ref, mvi_ref, age_ref, gdr_ref, occ_ref,
              tailt_ref, aget_ref, gdrt_ref, occt_ref, gw_ref, gb_ref,
              w1_ref, b1_ref, w2_ref, b2_ref, out_ref):
    f32 = jnp.float32
    W1 = w1_ref[...]
    Wa = W1[0:32]
    Wm = W1[32:64]
    Wg = W1[64:96]
    Wo = W1[96:128]
    Wgen = W1[128:160]

    def onehot_t(idx_row, n):
        iota = lax.broadcasted_iota(jnp.int32, (n, _B_BLK), 0)
        return (idx_row == iota).astype(f32)

    dot = functools.partial(jnp.dot, preferred_element_type=f32)

    def tdot(lhs_t, rhs):
        return lax.dot_general(
            lhs_t, rhs, (((0,), (0,)), ((), ())),
            preferred_element_type=f32,
        )

    mvi = mvi_ref[...]
    movT = jnp.where(mvi >= NROWB, 0.0, movT_ref[...])
    acc = tdot(movT, Wm)
    tail_iota = lax.broadcasted_iota(jnp.int32, (NTAIL, _B_BLK), 0)
    tail_oh = (mvi - NROWB == tail_iota).astype(f32)
    acc += tdot(tail_oh, dot(tailt_ref[...], Wm))
    acc += tdot(onehot_t(age_ref[...], NUM_AGE), dot(aget_ref[...], Wa))
    acc += tdot(onehot_t(gdr_ref[...], NUM_GENDERS), dot(gdrt_ref[...], Wg))
    acc += tdot(onehot_t(occ_ref[...], NUM_OCC), dot(occt_ref[...], Wo))
    acc += tdot(genT_ref[...], dot(gw_ref[...], Wgen))
    acc += dot(gb_ref[...], Wgen) + b1_ref[...]
    h = jnp.maximum(acc, 0.0)
    out_ref[...] = lax.dot_general(
        w2_ref[...], h, (((1,), (1,)), ((), ())),
        preferred_element_type=f32,
    ) + b2_ref[...]


def _mlp(movT, genresT, mvi_r, age_r, gdr_r, occ_r,
         tail_table, age_table, gender_table, occupation_table,
         genre_W, genre_b, fc1_W, fc1_b, fc2_W, fc2_b,
         interpret=False):
    row_spec = lambda d: pl.BlockSpec((d, _B_BLK), lambda i: (0, i))
    full_spec = lambda a, b: pl.BlockSpec((a, b), lambda i: (0, 0))
    out2 = pl.pallas_call(
        _mlp_body,
        grid=(_GRID,),
        in_specs=[
            row_spec(EMB),
            row_spec(NUM_GENRES),
            row_spec(1),
            row_spec(1),
            row_spec(1),
            row_spec(1),
            full_spec(NTAIL, EMB),
            full_spec(NUM_AGE, EMB),
            full_spec(NUM_GENDERS, EMB),
            full_spec(NUM_OCC, EMB),
            full_spec(NUM_GENRES, EMB),
            full_spec(1, EMB),
            full_spec(5 * EMB, HID),
            full_spec(1, HID),
            full_spec(1, HID),
            full_spec(1, 1),
        ],
        out_specs=pl.BlockSpec((1, _B_BLK), lambda i: (0, i)),
        out_shape=jax.ShapeDtypeStruct((1, BATCH), jnp.float32),
        interpret=interpret,
    )(movT, genresT, mvi_r, age_r, gdr_r, occ_r,
      tail_table, age_table, gender_table, occupation_table,
      genre_W, genre_b.reshape(1, EMB),
      fc1_W, fc1_b.reshape(1, HID), fc2_W.reshape(1, HID),
      fc2_b.reshape(1, 1))
    return out2.reshape(BATCH)


def kernel(age_group, movie, gender, occupation, genres,
           movie_table, gender_table, occupation_table, age_table,
           genre_W, genre_b, fc1_W, fc1_b, fc2_W, fc2_b):
    i32 = jnp.int32
    movie = movie.astype(i32)
    flat = _flatten(movie_table.T)
    movie_c = jnp.minimum(movie, NROWB - 1)
    idx_elems = (jnp.arange(EMB, dtype=i32) * NROWB)[:, None] + movie_c[None, :]
    movT = _movie_gather(flat, idx_elems)
    tail_table = lax.slice(movie_table, (NROWB, 0), (NROW, EMB))
    return _mlp(
        movT,
        genres.astype(jnp.float32).T,
        movie.reshape(1, BATCH),
        age_group.astype(i32).reshape(1, BATCH),
        gender.astype(i32).reshape(1, BATCH),
        occupation.astype(i32).reshape(1, BATCH),
        tail_table, age_table, gender_table, occupation_table,
        genre_W, genre_b, fc1_W, fc1_b, fc2_W, fc2_b,
    )


# speedup vs baseline: 17.3611x; 10.4573x over previous
import functools

import jax
import jax.numpy as jnp
from jax import lax
from jax.experimental import pallas as pl
from jax.experimental.pallas import tpu as pltpu
from jax.experimental.pallas import tpu_sc as plsc

EMB = 32
NROW = 1000000
NROWB = 999936
NTAIL = NROW - NROWB
BATCH = 16384
NUM_AGE = 7
NUM_GENDERS = 2
NUM_OCC = 21
NUM_GENRES = 18
HID = 128

_NC = 2
_NS = 16
_NW = _NC * _NS
_CHUNK = 128


_CP_STEPS = 42
_CP_L = NROWB // _CP_STEPS
_CP_R = 8 * _CP_L // 128
_TROW = 8 * NROWB


def _flatten_body(tblT_ref, out_ref):
    out_ref[...] = tblT_ref[...].reshape(_CP_R, 128)


def _flatten(tableT):
    out = pl.pallas_call(
        _flatten_body,
        grid=(EMB // 8, _CP_STEPS),
        in_specs=[pl.BlockSpec((8, _CP_L), lambda tr, c: (tr, c))],
        out_specs=pl.BlockSpec(
            (_CP_R, 128), lambda tr, c: (tr * _CP_STEPS + c, 0)
        ),
        out_shape=jax.ShapeDtypeStruct((EMB * NROWB // 128, 128), jnp.float32),
    )(tableT)
    return out.reshape(EMB * NROWB)


def _gather_body(flat_hbm, idx_hbm, outT_hbm, idx_v, vals_v, sem):
    wid = lax.axis_index("s") * _NC + lax.axis_index("c")
    pltpu.sync_copy(idx_hbm.at[wid], idx_v)
    copies = [
        pltpu.async_copy(
            flat_hbm.at[idx_v.at[pl.ds(j * _CHUNK, _CHUNK)]],
            vals_v.at[pl.ds(j * _CHUNK, _CHUNK)],
            sem,
        )
        for j in range(BATCH // _CHUNK)
    ]
    for c in copies:
        c.wait()
    pltpu.sync_copy(vals_v, outT_hbm.at[wid])


def _movie_gather(flat, idx_elems):
    mesh = plsc.VectorSubcoreMesh(core_axis_name="c", subcore_axis_name="s")
    k = pl.kernel(
        _gather_body,
        mesh=mesh,
        out_type=jax.ShapeDtypeStruct((EMB, BATCH), jnp.float32),
        scratch_types=[
            pltpu.VMEM((BATCH,), jnp.int32),
            pltpu.VMEM((BATCH,), jnp.float32),
            pltpu.SemaphoreType.DMA,
        ],
    )
    return k(flat, idx_elems)


_B_BLK = 2048
_GRID = BATCH // _B_BLK


def _mlp_body(movT_ref, genT_ref, mvi_ref, age_ref, gdr_ref, occ_ref,
              tailt_ref, aget_ref, gdrt_ref, occt_ref, gw_ref, gb_ref,
              w1_ref, b1_ref, w2_ref, b2_ref, out_ref):
    f32 = jnp.float32
    W1 = w1_ref[...]
    Wa = W1[0:32]
    Wm = W1[32:64]
    Wg = W1[64:96]
    Wo = W1[96:128]
    Wgen = W1[128:160]

    def onehot_t(idx_row, n):
        iota = lax.broadcasted_iota(jnp.int32, (n, _B_BLK), 0)
        return (idx_row == iota).astype(f32)

    dot = functools.partial(jnp.dot, preferred_element_type=f32)

    def tdot(lhs_t, rhs):
        return lax.dot_general(
            lhs_t, rhs, (((0,), (0,)), ((), ())),
            preferred_element_type=f32,
        )

    mvi = mvi_ref[...]
    movT = jnp.where(mvi >= NROWB, 0.0, movT_ref[...])
    acc = tdot(movT, Wm)
    tail_iota = lax.broadcasted_iota(jnp.int32, (NTAIL, _B_BLK), 0)
    tail_oh = (mvi - NROWB == tail_iota).astype(f32)
    acc += tdot(tail_oh, dot(tailt_ref[...], Wm))
    acc += tdot(onehot_t(age_ref[...], NUM_AGE), dot(aget_ref[...], Wa))
    acc += tdot(onehot_t(gdr_ref[...], NUM_GENDERS), dot(gdrt_ref[...], Wg))
    acc += tdot(onehot_t(occ_ref[...], NUM_OCC), dot(occt_ref[...], Wo))
    acc += tdot(genT_ref[...], dot(gw_ref[...], Wgen))
    acc += dot(gb_ref[...], Wgen) + b1_ref[...]
    h = jnp.maximum(acc, 0.0)
    out_ref[...] = lax.dot_general(
        w2_ref[...], h, (((1,), (1,)), ((), ())),
        preferred_element_type=f32,
    ) + b2_ref[...]


def _mlp(movT, genresT, mvi_r, age_r, gdr_r, occ_r,
         tail_table, age_table, gender_table, occupation_table,
         genre_W, genre_b, fc1_W, fc1_b, fc2_W, fc2_b,
         interpret=False):
    row_spec = lambda d: pl.BlockSpec((d, _B_BLK), lambda i: (0, i))
    full_spec = lambda a, b: pl.BlockSpec((a, b), lambda i: (0, 0))
    out2 = pl.pallas_call(
        _mlp_body,
        grid=(_GRID,),
        in_specs=[
            row_spec(EMB),
            row_spec(NUM_GENRES),
            row_spec(1),
            row_spec(1),
            row_spec(1),
            row_spec(1),
            full_spec(NTAIL, EMB),
            full_spec(NUM_AGE, EMB),
            full_spec(NUM_GENDERS, EMB),
            full_spec(NUM_OCC, EMB),
            full_spec(NUM_GENRES, EMB),
            full_spec(1, EMB),
            full_spec(5 * EMB, HID),
            full_spec(1, HID),
            full_spec(1, HID),
            full_spec(1, 1),
        ],
        out_specs=pl.BlockSpec((1, _B_BLK), lambda i: (0, i)),
        out_shape=jax.ShapeDtypeStruct((1, BATCH), jnp.float32),
        interpret=interpret,
    )(movT, genresT, mvi_r, age_r, gdr_r, occ_r,
      tail_table, age_table, gender_table, occupation_table,
      genre_W, genre_b.reshape(1, EMB),
      fc1_W, fc1_b.reshape(1, HID), fc2_W.reshape(1, HID),
      fc2_b.reshape(1, 1))
    return out2.reshape(BATCH)


def kernel(age_group, movie, gender, occupation, genres,
           movie_table, gender_table, occupation_table, age_table,
           genre_W, genre_b, fc1_W, fc1_b, fc2_W, fc2_b):
    i32 = jnp.int32
    movie = movie.astype(i32)
    flat = _flatten(movie_table.T)
    movie_c = jnp.minimum(movie, NROWB - 1)
    dd = jnp.arange(EMB, dtype=i32)[:, None]
    jj = movie_c[None, :]
    idx_elems = ((dd // 8) * _TROW + (jj // _CP_L) * (8 * _CP_L)
                 + (dd % 8) * _CP_L + (jj % _CP_L))
    movT = _movie_gather(flat, idx_elems)
    tail_table = lax.slice(movie_table, (NROWB, 0), (NROW, EMB))
    return _mlp(
        movT,
        genres.astype(jnp.float32).T,
        movie.reshape(1, BATCH),
        age_group.astype(i32).reshape(1, BATCH),
        gender.astype(i32).reshape(1, BATCH),
        occupation.astype(i32).reshape(1, BATCH),
        tail_table, age_table, gender_table, occupation_table,
        genre_W, genre_b, fc1_W, fc1_b, fc2_W, fc2_b,
    )


# speedup vs baseline: 18.3297x; 1.0558x over previous
import functools

import jax
import jax.numpy as jnp
from jax import lax
from jax.experimental import pallas as pl
from jax.experimental.pallas import tpu as pltpu
from jax.experimental.pallas import tpu_sc as plsc

EMB = 32
NROW = 1000000
NROWB = 999936
NTAIL = NROW - NROWB
BATCH = 16384
NUM_AGE = 7
NUM_GENDERS = 2
NUM_OCC = 21
NUM_GENRES = 18
HID = 128

_NC = 2
_NS = 16
_NW = _NC * _NS
_CHUNK = 128


_CP_STEPS = 42
_CP_L = NROWB // _CP_STEPS
_CP_R = 8 * _CP_L // 128
_TROW = 8 * NROWB


def _flatten_body(tblT_ref, out_ref):
    x = tblT_ref[...].reshape(8, _CP_L // 128, 128)
    out_ref[...] = x.swapaxes(0, 1).reshape(_CP_R, 128)


def _flatten(tableT):
    out = pl.pallas_call(
        _flatten_body,
        grid=(EMB // 8, _CP_STEPS),
        in_specs=[pl.BlockSpec((8, _CP_L), lambda tr, c: (tr, c))],
        out_specs=pl.BlockSpec(
            (_CP_R, 128), lambda tr, c: (tr * _CP_STEPS + c, 0)
        ),
        out_shape=jax.ShapeDtypeStruct((EMB * NROWB // 128, 128), jnp.float32),
    )(tableT)
    return out.reshape(EMB * NROWB)


def _gather_body(flat_hbm, idx_hbm, outT_hbm, idx_v, vals_v, sem):
    wid = lax.axis_index("s") * _NC + lax.axis_index("c")
    pltpu.sync_copy(idx_hbm.at[wid], idx_v)
    copies = [
        pltpu.async_copy(
            flat_hbm.at[idx_v.at[pl.ds(j * _CHUNK, _CHUNK)]],
            vals_v.at[pl.ds(j * _CHUNK, _CHUNK)],
            sem,
        )
        for j in range(BATCH // _CHUNK)
    ]
    for c in copies:
        c.wait()
    pltpu.sync_copy(vals_v, outT_hbm.at[wid])


def _movie_gather(flat, idx_elems):
    mesh = plsc.VectorSubcoreMesh(core_axis_name="c", subcore_axis_name="s")
    k = pl.kernel(
        _gather_body,
        mesh=mesh,
        out_type=jax.ShapeDtypeStruct((EMB, BATCH), jnp.float32),
        scratch_types=[
            pltpu.VMEM((BATCH,), jnp.int32),
            pltpu.VMEM((BATCH,), jnp.float32),
            pltpu.SemaphoreType.DMA,
        ],
    )
    return k(flat, idx_elems)


_B_BLK = 2048
_GRID = BATCH // _B_BLK


def _mlp_body(movT_ref, genT_ref, mvi_ref, age_ref, gdr_ref, occ_ref,
              tailt_ref, aget_ref, gdrt_ref, occt_ref, gw_ref, gb_ref,
              w1_ref, b1_ref, w2_ref, b2_ref, out_ref):
    f32 = jnp.float32
    W1 = w1_ref[...]
    Wa = W1[0:32]
    Wm = W1[32:64]
    Wg = W1[64:96]
    Wo = W1[96:128]
    Wgen = W1[128:160]

    def onehot_t(idx_row, n):
        iota = lax.broadcasted_iota(jnp.int32, (n, _B_BLK), 0)
        return (idx_row == iota).astype(f32)

    dot = functools.partial(jnp.dot, preferred_element_type=f32)

    def tdot(lhs_t, rhs):
        return lax.dot_general(
            lhs_t, rhs, (((0,), (0,)), ((), ())),
            preferred_element_type=f32,
        )

    mvi = mvi_ref[...]
    movT = jnp.where(mvi >= NROWB, 0.0, movT_ref[...])
    acc = tdot(movT, Wm)
    tail_iota = lax.broadcasted_iota(jnp.int32, (NTAIL, _B_BLK), 0)
    tail_oh = (mvi - NROWB == tail_iota).astype(f32)
    acc += tdot(tail_oh, dot(tailt_ref[...], Wm))
    acc += tdot(onehot_t(age_ref[...], NUM_AGE), dot(aget_ref[...], Wa))
    acc += tdot(onehot_t(gdr_ref[...], NUM_GENDERS), dot(gdrt_ref[...], Wg))
    acc += tdot(onehot_t(occ_ref[...], NUM_OCC), dot(occt_ref[...], Wo))
    acc += tdot(genT_ref[...], dot(gw_ref[...], Wgen))
    acc += dot(gb_ref[...], Wgen) + b1_ref[...]
    h = jnp.maximum(acc, 0.0)
    out_ref[...] = lax.dot_general(
        w2_ref[...], h, (((1,), (1,)), ((), ())),
        preferred_element_type=f32,
    ) + b2_ref[...]


def _mlp(movT, genresT, mvi_r, age_r, gdr_r, occ_r,
         tail_table, age_table, gender_table, occupation_table,
         genre_W, genre_b, fc1_W, fc1_b, fc2_W, fc2_b,
         interpret=False):
    row_spec = lambda d: pl.BlockSpec((d, _B_BLK), lambda i: (0, i))
    full_spec = lambda a, b: pl.BlockSpec((a, b), lambda i: (0, 0))
    out2 = pl.pallas_call(
        _mlp_body,
        grid=(_GRID,),
        in_specs=[
            row_spec(EMB),
            row_spec(NUM_GENRES),
            row_spec(1),
            row_spec(1),
            row_spec(1),
            row_spec(1),
            full_spec(NTAIL, EMB),
            full_spec(NUM_AGE, EMB),
            full_spec(NUM_GENDERS, EMB),
            full_spec(NUM_OCC, EMB),
            full_spec(NUM_GENRES, EMB),
            full_spec(1, EMB),
            full_spec(5 * EMB, HID),
            full_spec(1, HID),
            full_spec(1, HID),
            full_spec(1, 1),
        ],
        out_specs=pl.BlockSpec((1, _B_BLK), lambda i: (0, i)),
        out_shape=jax.ShapeDtypeStruct((1, BATCH), jnp.float32),
        interpret=interpret,
    )(movT, genresT, mvi_r, age_r, gdr_r, occ_r,
      tail_table, age_table, gender_table, occupation_table,
      genre_W, genre_b.reshape(1, EMB),
      fc1_W, fc1_b.reshape(1, HID), fc2_W.reshape(1, HID),
      fc2_b.reshape(1, 1))
    return out2.reshape(BATCH)


def kernel(age_group, movie, gender, occupation, genres,
           movie_table, gender_table, occupation_table, age_table,
           genre_W, genre_b, fc1_W, fc1_b, fc2_W, fc2_b):
    i32 = jnp.int32
    movie = movie.astype(i32)
    flat = _flatten(movie_table.T)
    movie_c = jnp.minimum(movie, NROWB - 1)
    dd = jnp.arange(EMB, dtype=i32)[:, None]
    jj = movie_c[None, :]
    idx_elems = ((dd // 8) * _TROW + (jj // _CP_L) * (8 * _CP_L)
                 + ((jj % _CP_L) // 128) * 1024
                 + (dd % 8) * 128 + (jj % 128))
    movT = _movie_gather(flat, idx_elems)
    tail_table = lax.slice(movie_table, (NROWB, 0), (NROW, EMB))
    return _mlp(
        movT,
        genres.astype(jnp.float32).T,
        movie.reshape(1, BATCH),
        age_group.astype(i32).reshape(1, BATCH),
        gender.astype(i32).reshape(1, BATCH),
        occupation.astype(i32).reshape(1, BATCH),
        tail_table, age_table, gender_table, occupation_table,
        genre_W, genre_b, fc1_W, fc1_b, fc2_W, fc2_b,
    )
